# Initial kernel scaffold; baseline (speedup 1.0000x reference)
import jax
import jax.numpy as jnp
from jax.experimental import pallas as pl


def kernel(x, hyperedges, hyperedge_attr, W0, att0, b0, W1, att1, b1):
    raise NotImplementedError("write your pallas kernel here")


# speedup vs baseline: 95.4442x; 95.4442x over previous
import functools

import jax
import jax.numpy as jnp
from jax import lax
from jax.experimental import pallas as pl
from jax.experimental.pallas import tpu as pltpu
from jax.experimental.pallas import tpu_sc as plsc

N_NODES = 50000
N_HE = 50000
NNZ = 800000
HEADS = 2
OUT = 32
D_IN = 64

NC = 2
NS = 16
LN = 16

BW = 80
IDX_ROWS = NNZ // BW
ROWS_PER_TILE = IDX_ROWS // NS
CHUNK_ROWS = 25
EDGES_PER_CHUNK = CHUNK_ROWS * BW
NCHUNKS = ROWS_PER_TILE // CHUNK_ROWS

SEG_FULL = 3136
SEG_LAST = N_HE - 15 * SEG_FULL
OTILE = 3200
OCH = 400
OSIZES_FULL = (400,) * 8
OSIZES_LAST = (400,) * 5
CHB = 5
EPB = CHB * BW
NCHB = ROWS_PER_TILE // CHB

_EPS = 1e-16
_f32 = jnp.float32


def _seg_range(s):
    return s * SEG_FULL


def _tc_embed(F, he_attr, W, Ai, Aj):
    n = F.shape[0]
    blk = 1000
    grid = n // blk

    def body(f_ref, he_ref, w_ref, ai_w_ref, aj_w_ref,
             xh_ref, ai_ref, aj_ref, m4_ref, msc):
        b = pl.program_id(0)
        xh = jnp.dot(f_ref[...], w_ref[...], preferred_element_type=_f32)
        heh = jnp.dot(he_ref[...], w_ref[...], preferred_element_type=_f32)
        ai = jnp.dot(xh, ai_w_ref[...], preferred_element_type=_f32)
        aj = jnp.dot(heh, aj_w_ref[...], preferred_element_type=_f32)
        xh_ref[0] = xh[:, :OUT]
        xh_ref[1] = xh[:, OUT:]
        ai_ref[...] = ai
        aj_ref[...] = aj
        mi0 = jnp.max(ai[:, 0])
        mi1 = jnp.max(ai[:, 1])
        mj0 = jnp.max(aj[:, 0])
        mj1 = jnp.max(aj[:, 1])

        @pl.when(b == 0)
        def _():
            msc[0], msc[1], msc[2], msc[3] = mi0, mi1, mj0, mj1

        @pl.when(b > 0)
        def _():
            msc[0] = jnp.maximum(msc[0], mi0)
            msc[1] = jnp.maximum(msc[1], mi1)
            msc[2] = jnp.maximum(msc[2], mj0)
            msc[3] = jnp.maximum(msc[3], mj1)

        @pl.when(b == pl.num_programs(0) - 1)
        def _():
            m4_ref[0], m4_ref[1] = msc[0], msc[1]
            m4_ref[2], m4_ref[3] = msc[2], msc[3]

    return pl.pallas_call(
        body,
        grid=(grid,),
        in_specs=[
            pl.BlockSpec((blk, D_IN), lambda b: (b, 0)),
            pl.BlockSpec((blk, D_IN), lambda b: (b, 0)),
            pl.BlockSpec((D_IN, D_IN), lambda b: (0, 0)),
            pl.BlockSpec((D_IN, HEADS), lambda b: (0, 0)),
            pl.BlockSpec((D_IN, HEADS), lambda b: (0, 0)),
        ],
        out_specs=[
            pl.BlockSpec((HEADS, blk, OUT), lambda b: (0, b, 0)),
            pl.BlockSpec((blk, HEADS), lambda b: (b, 0)),
            pl.BlockSpec((blk, HEADS), lambda b: (b, 0)),
            pl.BlockSpec(memory_space=pltpu.SMEM),
        ],
        out_shape=[
            jax.ShapeDtypeStruct((HEADS, n, OUT), _f32),
            jax.ShapeDtypeStruct((n, HEADS), _f32),
            jax.ShapeDtypeStruct((he_attr.shape[0], HEADS), _f32),
            jax.ShapeDtypeStruct((4,), _f32),
        ],
        scratch_shapes=[pltpu.SMEM((4,), _f32)],
    )(F, he_attr, W, Ai, Aj)


def _zero_1d(buf, nwords):
    def zb(k, _):
        buf[pl.ds(k * LN, LN)] = jnp.zeros((LN,), _f32)
        return _
    lax.fori_loop(0, nwords // LN, zb, None)


def _make_sc_a(first_layer):
    mesh = plsc.VectorSubcoreMesh(core_axis_name="c", subcore_axis_name="s",
                                  num_cores=NC, num_subcores=NS)

    out_type = [
        jax.ShapeDtypeStruct((HEADS, NNZ), _f32),
        jax.ShapeDtypeStruct((HEADS, NNZ), _f32),
        jax.ShapeDtypeStruct((HEADS, NNZ), _f32),
    ]
    if first_layer:
        out_type += [
            jax.ShapeDtypeStruct((N_NODES,), _f32),
            jax.ShapeDtypeStruct((N_HE,), _f32),
        ]

    scratch = dict(
        tbl_a=pltpu.VMEM((N_NODES,), _f32),
        tbl_b=pltpu.VMEM((N_HE,), _f32),
        sidx=pltpu.VMEM((CHUNK_ROWS, BW), jnp.int32),
        didx=pltpu.VMEM((CHUNK_ROWS, BW), jnp.int32),
        ebuf=pltpu.VMEM((EDGES_PER_CHUNK,), _f32),
        a1buf=pltpu.VMEM((EDGES_PER_CHUNK,), _f32),
        ones=pltpu.VMEM((BW,), _f32),
        mbuf=pltpu.VMEM((LN,), _f32),
        d1=pltpu.VMEM((SEG_FULL,), _f32),
        d2=pltpu.VMEM((SEG_FULL,), _f32),
        d3=pltpu.VMEM((SEG_FULL,), _f32),
        den_sp=pltpu.VMEM_SHARED((N_HE,), _f32),
        cdst_sp=pltpu.VMEM_SHARED((N_HE,), _f32),
        csrc_sp=pltpu.VMEM_SHARED((N_NODES,), _f32),
    )

    def body(src2d, dst2d, ai_hm, aj_hm, mh, cntdst_in,
             ex_o, alpha_o, alpha1_o, drec_o, cntdst_o,
             tbl_a, tbl_b, sidx, didx, ebuf, a1buf, ones, mbuf,
             d1, d2, d3, den_sp, cdst_sp, csrc_sp):
        c = lax.axis_index("c")
        s = lax.axis_index("s")

        _zero_1d(ebuf, EDGES_PER_CHUNK)
        for t in range(BW // LN):
            ones[pl.ds(t * LN, LN)] = jnp.ones((LN,), _f32)
        st = s * SEG_FULL

        def zero_seg(sp_ref, n0, n1):
            pltpu.sync_copy(ebuf.at[pl.ds(0, n0)], sp_ref.at[pl.ds(st, n0)])
            pltpu.sync_copy(ebuf.at[pl.ds(0, n1)],
                            sp_ref.at[pl.ds(st + n0, n1)])

        for sp in ([den_sp, cdst_sp, csrc_sp] if first_layer else [den_sp]):
            @pl.when(s < NS - 1)
            def _(sp=sp):
                zero_seg(sp, 2000, SEG_FULL - 2000)

            @pl.when(s == NS - 1)
            def _(sp=sp):
                zero_seg(sp, 2000, SEG_LAST - 2000)
        plsc.subcore_barrier()

        pltpu.sync_copy(ai_hm.at[c], tbl_a)
        pltpu.sync_copy(aj_hm.at[c], tbl_b)
        pltpu.sync_copy(mh.at[c], mbuf)
        mvec = mbuf[...]

        def p1_chunk(ch, _):
            rb = s * ROWS_PER_TILE + ch * CHUNK_ROWS
            pltpu.sync_copy(src2d.at[pl.ds(rb, CHUNK_ROWS)], sidx)
            pltpu.sync_copy(dst2d.at[pl.ds(rb, CHUNK_ROWS)], didx)

            def row(j, _):
                for t in range(BW // LN):
                    iv = sidx[j, pl.ds(t * LN, LN)]
                    jv = didx[j, pl.ds(t * LN, LN)]
                    av = plsc.load_gather(tbl_a, [iv])
                    bv = plsc.load_gather(tbl_b, [jv])
                    logit = av + bv
                    logit = jnp.maximum(logit, 0.2 * logit)
                    ex = jnp.exp(logit - mvec)
                    ebuf[pl.ds(j * BW + t * LN, LN)] = ex
                return _
            lax.fori_loop(0, CHUNK_ROWS, row, None)

            pltpu.sync_copy(ebuf, ex_o.at[c, pl.ds(rb * BW, EDGES_PER_CHUNK)])

            for j in range(CHUNK_ROWS):
                pltpu.sync_copy(ebuf.at[pl.ds(j * BW, BW)],
                                den_sp.at[didx.at[j]], add=True)
                if first_layer:
                    pltpu.sync_copy(ones, cdst_sp.at[didx.at[j]], add=True)
                    pltpu.sync_copy(ones, csrc_sp.at[sidx.at[j]], add=True)
            return _
        lax.fori_loop(0, NCHUNKS, p1_chunk, None)
        plsc.subcore_barrier()

        def p15(size):
            pltpu.sync_copy(den_sp.at[pl.ds(st, size)], d1.at[pl.ds(0, size)])
            if first_layer:
                pltpu.sync_copy(cdst_sp.at[pl.ds(st, size)],
                                d2.at[pl.ds(0, size)])
                pltpu.sync_copy(csrc_sp.at[pl.ds(st, size)],
                                d3.at[pl.ds(0, size)])

                @pl.when(c == 1)
                def _():
                    pltpu.sync_copy(d2.at[pl.ds(0, size)],
                                    cntdst_o.at[pl.ds(st, size)])
            else:
                pltpu.sync_copy(cntdst_in.at[pl.ds(st, size)],
                                d2.at[pl.ds(0, size)])

            def vec(k, _):
                sl = pl.ds(k * LN, LN)
                de = d1[sl] + _EPS
                d1[sl] = de
                d2[sl] = de * d2[sl]
                if first_layer:
                    cs = d3[sl]
                    d3[sl] = jnp.where(cs > 0.0, 1.0 / cs, 0.0)
                return _
            lax.fori_loop(0, size // LN, vec, None)

            pltpu.sync_copy(d1.at[pl.ds(0, size)], den_sp.at[pl.ds(st, size)])
            pltpu.sync_copy(d2.at[pl.ds(0, size)], cdst_sp.at[pl.ds(st, size)])
            if first_layer:
                @pl.when(c == 0)
                def _():
                    pltpu.sync_copy(d3.at[pl.ds(0, size)],
                                    drec_o.at[pl.ds(st, size)])

        @pl.when(s < NS - 1)
        def _():
            p15(SEG_FULL)

        @pl.when(s == NS - 1)
        def _():
            p15(SEG_LAST)
        plsc.subcore_barrier()

        pltpu.sync_copy(den_sp, tbl_a)
        pltpu.sync_copy(cdst_sp, tbl_b)

        def p2_chunk(ch, _):
            rb = s * ROWS_PER_TILE + ch * CHUNK_ROWS
            eb = rb * BW
            pltpu.sync_copy(dst2d.at[pl.ds(rb, CHUNK_ROWS)], didx)
            pltpu.sync_copy(ex_o.at[c, pl.ds(eb, EDGES_PER_CHUNK)], ebuf)

            def row(j, _):
                for t in range(BW // LN):
                    sl = pl.ds(j * BW + t * LN, LN)
                    jv = didx[j, pl.ds(t * LN, LN)]
                    exv = ebuf[sl]
                    d0 = plsc.load_gather(tbl_a, [jv])
                    dd1 = plsc.load_gather(tbl_b, [jv])
                    ebuf[sl] = exv / d0
                    a1buf[sl] = exv / dd1
                return _
            lax.fori_loop(0, CHUNK_ROWS, row, None)
            pltpu.sync_copy(ebuf, alpha_o.at[c, pl.ds(eb, EDGES_PER_CHUNK)])
            pltpu.sync_copy(a1buf, alpha1_o.at[c, pl.ds(eb, EDGES_PER_CHUNK)])
            return _
        lax.fori_loop(0, NCHUNKS, p2_chunk, None)

    cparams = pltpu.CompilerParams(use_tc_tiling_on_sc=False, needs_layout_passes=False)
    if first_layer:
        def call(src2d, dst2d, ai_hm, aj_hm, mh):
            def body0(src2d, dst2d, ai_hm, aj_hm, mh, *rest):
                return body(src2d, dst2d, ai_hm, aj_hm, mh, None, *rest)
            fn = pl.kernel(body0, out_type=out_type, mesh=mesh,
                           scratch_types=list(scratch.values()),
                           compiler_params=cparams)
            return fn(src2d, dst2d, ai_hm, aj_hm, mh)
        return call
    else:
        def call(src2d, dst2d, ai_hm, aj_hm, mh, cntdst):
            def body1(src2d, dst2d, ai_hm, aj_hm, mh, cntdst_in,
                      ex_o, alpha_o, alpha1_o, *rest):
                return body(src2d, dst2d, ai_hm, aj_hm, mh, cntdst_in,
                            ex_o, alpha_o, alpha1_o, None, None, *rest)
            fn = pl.kernel(body1, out_type=out_type, mesh=mesh,
                           scratch_types=list(scratch.values()),
                           compiler_params=cparams)
            return fn(src2d, dst2d, ai_hm, aj_hm, mh, cntdst)
        return call


def _make_sc_b(first_layer):
    mesh = plsc.VectorSubcoreMesh(core_axis_name="c", subcore_axis_name="s",
                                  num_cores=NC, num_subcores=NS)
    out_type = [
        jax.ShapeDtypeStruct((HEADS, N_NODES, OUT), _f32),
        jax.ShapeDtypeStruct((HEADS, N_HE, OUT), _f32),
    ]
    scratch = [
        pltpu.VMEM((2 * EPB, OUT), _f32),
        pltpu.VMEM((CHB, BW), jnp.int32),
        pltpu.VMEM((CHB, BW), jnp.int32),
        pltpu.VMEM((EPB,), _f32),
        pltpu.VMEM((OCH,), _f32),
        pltpu.VMEM((OUT,), _f32),
        pltpu.VMEM_SHARED((N_HE, OUT), _f32),
        pltpu.SemaphoreType.DMA,
    ]

    def body(src2d, dst2d, xh_hm, alpha, alpha1, drec, bias_hm,
             h_o, oute_o,
             rowbuf, sidx, didx, abuf, dbuf, biasv, acc, sem):
        c = lax.axis_index("c")
        s = lax.axis_index("s")

        def zrows(r0, nrows):
            def zr(r, _):
                rowbuf[r, pl.ds(0, LN)] = jnp.zeros((LN,), _f32)
                rowbuf[r, pl.ds(LN, LN)] = jnp.zeros((LN,), _f32)
                return _
            lax.fori_loop(r0, r0 + nrows, zr, None)
        zrows(0, 2 * EPB)
        st = s * SEG_FULL

        def zseg(size3):
            o = 0
            for nsz in size3:
                pltpu.sync_copy(rowbuf.at[pl.ds(0, nsz)],
                                acc.at[pl.ds(st + o, nsz)])
                o += nsz

        @pl.when(s < NS - 1)
        def _():
            zseg((784, 784, 784, 784))

        @pl.when(s == NS - 1)
        def _():
            zseg((784, 784, 784, SEG_LAST - 3 * 784))
        plsc.subcore_barrier()

        def scale_rows(wbuf):
            def blk(q, _):
                for e16 in range(LN):
                    e = q * LN + e16
                    w = plsc.load_gather(wbuf, [jnp.full((LN,), e, jnp.int32)])
                    rowbuf[e, pl.ds(0, LN)] = rowbuf[e, pl.ds(0, LN)] * w
                    rowbuf[e, pl.ds(LN, LN)] = rowbuf[e, pl.ds(LN, LN)] * w
                return _
            lax.fori_loop(0, EPB // LN, blk, None)

        def propagate(table_hbm, gidx, scatteridx, wsrc):
            def chunk(ch, _):
                rb = s * ROWS_PER_TILE + ch * CHB
                pltpu.sync_copy(src2d.at[pl.ds(rb, CHB)], sidx)
                pltpu.sync_copy(dst2d.at[pl.ds(rb, CHB)], didx)
                pltpu.sync_copy(wsrc.at[c, pl.ds(rb * BW, EPB)], abuf)
                g = sidx if gidx == "src" else didx
                sc = sidx if scatteridx == "src" else didx
                cps = []
                for j in range(CHB):
                    cps.append(pltpu.async_copy(
                        table_hbm.at[c].at[g.at[j]],
                        rowbuf.at[pl.ds(j * BW, BW)], sem))
                for cp in cps:
                    cp.wait()
                scale_rows(abuf)

                for j in range(CHB):
                    pltpu.sync_copy(rowbuf.at[pl.ds(j * BW, BW)],
                                    acc.at[sc.at[j]], add=True)
                return _
            lax.fori_loop(0, NCHB, chunk, None)

        propagate(xh_hm, "src", "dst", alpha1)
        plsc.subcore_barrier()

        r0t = s * OTILE

        def wout(sizes):
            o = 0
            for nsz in sizes:
                r0 = r0t + o
                o += nsz
                pltpu.sync_copy(acc.at[pl.ds(r0, nsz)],
                                rowbuf.at[pl.ds(0, nsz)])
                pltpu.sync_copy(rowbuf.at[pl.ds(0, nsz)],
                                oute_o.at[c, pl.ds(r0, nsz)])
                pltpu.sync_copy(rowbuf.at[pl.ds(EPB, nsz)],
                                acc.at[pl.ds(r0, nsz)])

        @pl.when(s < NS - 1)
        def _():
            wout(OSIZES_FULL)

        @pl.when(s == NS - 1)
        def _():
            wout(OSIZES_LAST)
        plsc.subcore_barrier()

        propagate(oute_o, "dst", "src", alpha)
        plsc.subcore_barrier()

        pltpu.sync_copy(bias_hm.at[c], biasv)
        blo = biasv[pl.ds(0, LN)]
        bhi = biasv[pl.ds(LN, LN)]

        def fin(sizes):
            o = 0
            for nsz in sizes:
                r0 = r0t + o
                o += nsz
                pltpu.sync_copy(acc.at[pl.ds(r0, nsz)],
                                rowbuf.at[pl.ds(0, nsz)])
                pltpu.sync_copy(drec.at[pl.ds(r0, nsz)],
                                dbuf.at[pl.ds(0, nsz)])

                def frow(r, _):
                    dv = plsc.load_gather(dbuf,
                                          [jnp.full((LN,), r, jnp.int32)])
                    lo = rowbuf[r, pl.ds(0, LN)] * dv + blo
                    hi = rowbuf[r, pl.ds(LN, LN)] * dv + bhi
                    if first_layer:
                        lo = jnp.maximum(lo, 0.0)
                        hi = jnp.maximum(hi, 0.0)
                    rowbuf[r, pl.ds(0, LN)] = lo
                    rowbuf[r, pl.ds(LN, LN)] = hi
                    return _
                lax.fori_loop(0, nsz, frow, None)
                pltpu.sync_copy(rowbuf.at[pl.ds(0, nsz)],
                                h_o.at[c, pl.ds(r0, nsz)])

        @pl.when(s < NS - 1)
        def _():
            fin(OSIZES_FULL)

        @pl.when(s == NS - 1)
        def _():
            fin(OSIZES_LAST)

    return pl.kernel(body, out_type=out_type, mesh=mesh,
                     scratch_types=scratch,
                     compiler_params=pltpu.CompilerParams(
                         use_tc_tiling_on_sc=False,
                         needs_layout_passes=False))


def _att_mats(att):
    a = att[0]
    Ai = jnp.zeros((D_IN, HEADS), _f32)
    Aj = jnp.zeros((D_IN, HEADS), _f32)
    for h in range(HEADS):
        Ai = Ai.at[h * OUT:(h + 1) * OUT, h].set(a[h, :OUT])
        Aj = Aj.at[h * OUT:(h + 1) * OUT, h].set(a[h, OUT:])
    return Ai, Aj


@jax.jit
def kernel(x, hyperedges, hyperedge_attr, W0, att0, b0, W1, att1, b1):
    src2d = hyperedges[0].reshape(IDX_ROWS, BW)
    dst2d = hyperedges[1].reshape(IDX_ROWS, BW)

    sc_a0 = _make_sc_a(True)
    sc_a1 = _make_sc_a(False)
    sc_b0 = _make_sc_b(True)
    sc_b1 = _make_sc_b(False)

    def layer(F, W, att, b, sc_a, sc_b, extra):
        Ai, Aj = _att_mats(att)
        xh_hm, ai, aj, m4 = _tc_embed(F, hyperedge_attr, W, Ai, Aj)
        ai_hm = ai.T
        aj_hm = aj.T
        m = m4[:2] + m4[2:]
        m = jnp.maximum(m, 0.2 * m)
        mh = jnp.tile(m[:, None], (1, LN))
        if extra is None:
            _ex, alpha, alpha1, drec, cntdst = sc_a(
                src2d, dst2d, ai_hm, aj_hm, mh)
        else:
            drec, cntdst = extra
            _ex, alpha, alpha1 = sc_a(src2d, dst2d, ai_hm, aj_hm, mh, cntdst)
        bias_hm = b.reshape(HEADS, OUT)
        h_hm, _oute = sc_b(src2d, dst2d, xh_hm, alpha, alpha1, drec, bias_hm)
        return h_hm, (drec, cntdst)

    h_hm, dd = layer(x, W0, att0, b0, sc_a0, sc_b0, None)
    h64 = h_hm.transpose(1, 0, 2).reshape(N_NODES, HEADS * OUT)
    out_hm, _ = layer(h64, W1, att1, b1, sc_a1, sc_b1, dd)
    return out_hm.transpose(1, 0, 2).reshape(N_NODES, HEADS * OUT)


# speedup vs baseline: 165.0285x; 1.7291x over previous
import functools

import jax
import jax.numpy as jnp
from jax import lax
from jax.experimental import pallas as pl
from jax.experimental.pallas import tpu as pltpu
from jax.experimental.pallas import tpu_sc as plsc

N_NODES = 50000
N_HE = 50000
NNZ = 800000
HEADS = 2
OUT = 32
D_IN = 64

NC = 2
NS = 16
LN = 16

BW = 80
IDX_ROWS = NNZ // BW
ROWS_PER_TILE = IDX_ROWS // NS
CHUNK_ROWS = 25
EDGES_PER_CHUNK = CHUNK_ROWS * BW
NCHUNKS = ROWS_PER_TILE // CHUNK_ROWS

SEG_FULL = 3136
SEG_LAST = N_HE - 15 * SEG_FULL
OTILE = 3200
OLAST = N_NODES - 15 * OTILE
OCH = 400
CHB = 5
EPB = CHB * BW
NCHB = ROWS_PER_TILE // CHB

_EPS = 1e-16
_f32 = jnp.float32


def _seg_range(s):
    return s * SEG_FULL


def _tc_embed(F, he_attr, W, Ai, Aj):
    n = F.shape[0]
    blk = 1000
    grid = n // blk

    def body(f_ref, he_ref, w_ref, ai_w_ref, aj_w_ref,
             xh_ref, ai_ref, aj_ref, m4_ref, msc):
        b = pl.program_id(0)
        xh = jnp.dot(f_ref[...], w_ref[...], preferred_element_type=_f32)
        heh = jnp.dot(he_ref[...], w_ref[...], preferred_element_type=_f32)
        ai = jnp.dot(xh, ai_w_ref[...], preferred_element_type=_f32)
        aj = jnp.dot(heh, aj_w_ref[...], preferred_element_type=_f32)
        xh_ref[0] = xh[:, :OUT]
        xh_ref[1] = xh[:, OUT:]
        ai_ref[...] = ai
        aj_ref[...] = aj
        mi0 = jnp.max(ai[:, 0])
        mi1 = jnp.max(ai[:, 1])
        mj0 = jnp.max(aj[:, 0])
        mj1 = jnp.max(aj[:, 1])

        @pl.when(b == 0)
        def _():
            msc[0], msc[1], msc[2], msc[3] = mi0, mi1, mj0, mj1

        @pl.when(b > 0)
        def _():
            msc[0] = jnp.maximum(msc[0], mi0)
            msc[1] = jnp.maximum(msc[1], mi1)
            msc[2] = jnp.maximum(msc[2], mj0)
            msc[3] = jnp.maximum(msc[3], mj1)

        @pl.when(b == pl.num_programs(0) - 1)
        def _():
            m4_ref[0], m4_ref[1] = msc[0], msc[1]
            m4_ref[2], m4_ref[3] = msc[2], msc[3]

    return pl.pallas_call(
        body,
        grid=(grid,),
        in_specs=[
            pl.BlockSpec((blk, D_IN), lambda b: (b, 0)),
            pl.BlockSpec((blk, D_IN), lambda b: (b, 0)),
            pl.BlockSpec((D_IN, D_IN), lambda b: (0, 0)),
            pl.BlockSpec((D_IN, HEADS), lambda b: (0, 0)),
            pl.BlockSpec((D_IN, HEADS), lambda b: (0, 0)),
        ],
        out_specs=[
            pl.BlockSpec((HEADS, blk, OUT), lambda b: (0, b, 0)),
            pl.BlockSpec((blk, HEADS), lambda b: (b, 0)),
            pl.BlockSpec((blk, HEADS), lambda b: (b, 0)),
            pl.BlockSpec(memory_space=pltpu.SMEM),
        ],
        out_shape=[
            jax.ShapeDtypeStruct((HEADS, n, OUT), _f32),
            jax.ShapeDtypeStruct((n, HEADS), _f32),
            jax.ShapeDtypeStruct((he_attr.shape[0], HEADS), _f32),
            jax.ShapeDtypeStruct((4,), _f32),
        ],
        scratch_shapes=[pltpu.SMEM((4,), _f32)],
    )(F, he_attr, W, Ai, Aj)


def _zero_1d(buf, nwords):
    def zb(k, _):
        buf[pl.ds(k * LN, LN)] = jnp.zeros((LN,), _f32)
        return _
    lax.fori_loop(0, nwords // LN, zb, None)


def _make_sc_a(first_layer):
    mesh = plsc.VectorSubcoreMesh(core_axis_name="c", subcore_axis_name="s",
                                  num_cores=NC, num_subcores=NS)

    out_type = [
        jax.ShapeDtypeStruct((HEADS, NNZ), _f32),
        jax.ShapeDtypeStruct((HEADS, NNZ), _f32),
        jax.ShapeDtypeStruct((HEADS, NNZ), _f32),
    ]
    if first_layer:
        out_type += [
            jax.ShapeDtypeStruct((N_NODES,), _f32),
            jax.ShapeDtypeStruct((N_HE,), _f32),
        ]

    scratch = dict(
        tbl_a=pltpu.VMEM((N_NODES,), _f32),
        tbl_b=pltpu.VMEM((N_HE,), _f32),
        sidx=pltpu.VMEM((EDGES_PER_CHUNK,), jnp.int32),
        didx=pltpu.VMEM((EDGES_PER_CHUNK,), jnp.int32),
        ebuf=pltpu.VMEM((EDGES_PER_CHUNK,), _f32),
        a1buf=pltpu.VMEM((EDGES_PER_CHUNK,), _f32),
        ones=pltpu.VMEM((EDGES_PER_CHUNK,), _f32),
        mbuf=pltpu.VMEM((LN,), _f32),
        d1=pltpu.VMEM((SEG_FULL,), _f32),
        d2=pltpu.VMEM((SEG_FULL,), _f32),
        d3=pltpu.VMEM((SEG_FULL,), _f32),
        den_sp=pltpu.VMEM_SHARED((N_HE,), _f32),
        cdst_sp=pltpu.VMEM_SHARED((N_HE,), _f32),
        csrc_sp=pltpu.VMEM_SHARED((N_NODES,), _f32),
    )

    def body(src1, dst1, ai_hm, aj_hm, mh, cntdst_in,
             ex_o, alpha_o, alpha1_o, drec_o, cntdst_o,
             tbl_a, tbl_b, sidx, didx, ebuf, a1buf, ones, mbuf,
             d1, d2, d3, den_sp, cdst_sp, csrc_sp):
        c = lax.axis_index("c")
        s = lax.axis_index("s")

        def ffill(buf, n, val):
            def zb(k, _):
                buf[pl.ds(k * LN, LN)] = jnp.full((LN,), val, _f32)
                return _
            lax.fori_loop(0, n // LN, zb, None)
        ffill(ones, EDGES_PER_CHUNK, 1.0)
        ffill(d1, SEG_FULL, 0.0)
        st = s * SEG_FULL

        for sp in ([den_sp, cdst_sp, csrc_sp] if first_layer else [den_sp]):
            @pl.when(s < NS - 1)
            def _(sp=sp):
                pltpu.sync_copy(d1.at[pl.ds(0, SEG_FULL)],
                                sp.at[pl.ds(st, SEG_FULL)])

            @pl.when(s == NS - 1)
            def _(sp=sp):
                pltpu.sync_copy(d1.at[pl.ds(0, SEG_LAST)],
                                sp.at[pl.ds(st, SEG_LAST)])
        plsc.subcore_barrier()

        pltpu.sync_copy(ai_hm.at[c], tbl_a)
        pltpu.sync_copy(aj_hm.at[c], tbl_b)
        pltpu.sync_copy(mh.at[c], mbuf)
        mvec = mbuf[...]

        def p1_chunk(ch, _):
            eb = (s * ROWS_PER_TILE + ch * CHUNK_ROWS) * BW
            pltpu.sync_copy(src1.at[pl.ds(eb, EDGES_PER_CHUNK)], sidx)
            pltpu.sync_copy(dst1.at[pl.ds(eb, EDGES_PER_CHUNK)], didx)

            def vec(k, _):
                sl = pl.ds(k * LN, LN)
                av = plsc.load_gather(tbl_a, [sidx[sl]])
                bv = plsc.load_gather(tbl_b, [didx[sl]])
                logit = av + bv
                logit = jnp.maximum(logit, 0.2 * logit)
                ebuf[sl] = jnp.exp(logit - mvec)
                return _
            lax.fori_loop(0, EDGES_PER_CHUNK // LN, vec, None)

            pltpu.sync_copy(ebuf, ex_o.at[c, pl.ds(eb, EDGES_PER_CHUNK)])
            pltpu.sync_copy(ebuf, den_sp.at[didx], add=True)
            if first_layer:
                pltpu.sync_copy(ones, cdst_sp.at[didx], add=True)
                pltpu.sync_copy(ones, csrc_sp.at[sidx], add=True)
            return _
        lax.fori_loop(0, NCHUNKS, p1_chunk, None)
        plsc.subcore_barrier()

        def p15(size):
            pltpu.sync_copy(den_sp.at[pl.ds(st, size)], d1.at[pl.ds(0, size)])
            if first_layer:
                pltpu.sync_copy(cdst_sp.at[pl.ds(st, size)],
                                d2.at[pl.ds(0, size)])
                pltpu.sync_copy(csrc_sp.at[pl.ds(st, size)],
                                d3.at[pl.ds(0, size)])

                @pl.when(c == 1)
                def _():
                    pltpu.sync_copy(d2.at[pl.ds(0, size)],
                                    cntdst_o.at[pl.ds(st, size)])
            else:
                pltpu.sync_copy(cntdst_in.at[pl.ds(st, size)],
                                d2.at[pl.ds(0, size)])

            def vec(k, _):
                sl = pl.ds(k * LN, LN)
                de = d1[sl] + _EPS
                d1[sl] = de
                d2[sl] = de * d2[sl]
                if first_layer:
                    cs = d3[sl]
                    d3[sl] = jnp.where(cs > 0.0, 1.0 / cs, 0.0)
                return _
            lax.fori_loop(0, size // LN, vec, None)

            pltpu.sync_copy(d1.at[pl.ds(0, size)], den_sp.at[pl.ds(st, size)])
            pltpu.sync_copy(d2.at[pl.ds(0, size)], cdst_sp.at[pl.ds(st, size)])
            if first_layer:
                @pl.when(c == 0)
                def _():
                    pltpu.sync_copy(d3.at[pl.ds(0, size)],
                                    drec_o.at[pl.ds(st, size)])

        @pl.when(s < NS - 1)
        def _():
            p15(SEG_FULL)

        @pl.when(s == NS - 1)
        def _():
            p15(SEG_LAST)
        plsc.subcore_barrier()

        pltpu.sync_copy(den_sp, tbl_a)
        pltpu.sync_copy(cdst_sp, tbl_b)

        def p2_chunk(ch, _):
            eb = (s * ROWS_PER_TILE + ch * CHUNK_ROWS) * BW
            pltpu.sync_copy(dst1.at[pl.ds(eb, EDGES_PER_CHUNK)], didx)
            pltpu.sync_copy(ex_o.at[c, pl.ds(eb, EDGES_PER_CHUNK)], ebuf)

            def vec(k, _):
                sl = pl.ds(k * LN, LN)
                jv = didx[sl]
                exv = ebuf[sl]
                d0 = plsc.load_gather(tbl_a, [jv])
                dd1 = plsc.load_gather(tbl_b, [jv])
                ebuf[sl] = exv / d0
                a1buf[sl] = exv / dd1
                return _
            lax.fori_loop(0, EDGES_PER_CHUNK // LN, vec, None)
            pltpu.sync_copy(ebuf, alpha_o.at[c, pl.ds(eb, EDGES_PER_CHUNK)])
            pltpu.sync_copy(a1buf, alpha1_o.at[c, pl.ds(eb, EDGES_PER_CHUNK)])
            return _
        lax.fori_loop(0, NCHUNKS, p2_chunk, None)

    cparams = pltpu.CompilerParams(use_tc_tiling_on_sc=False,
                                   needs_layout_passes=False)
    if first_layer:
        def call(src1, dst1, ai_hm, aj_hm, mh):
            def body0(src1, dst1, ai_hm, aj_hm, mh, *rest):
                return body(src1, dst1, ai_hm, aj_hm, mh, None, *rest)
            fn = pl.kernel(body0, out_type=out_type, mesh=mesh,
                           scratch_types=list(scratch.values()),
                           compiler_params=cparams)
            return fn(src1, dst1, ai_hm, aj_hm, mh)
        return call
    else:
        def call(src1, dst1, ai_hm, aj_hm, mh, cntdst):
            def body1(src1, dst1, ai_hm, aj_hm, mh, cntdst_in,
                      ex_o, alpha_o, alpha1_o, *rest):
                return body(src1, dst1, ai_hm, aj_hm, mh, cntdst_in,
                            ex_o, alpha_o, alpha1_o, None, None, *rest)
            fn = pl.kernel(body1, out_type=out_type, mesh=mesh,
                           scratch_types=list(scratch.values()),
                           compiler_params=cparams)
            return fn(src1, dst1, ai_hm, aj_hm, mh, cntdst)
        return call


def _make_sc_b(first_layer):
    mesh = plsc.VectorSubcoreMesh(core_axis_name="c", subcore_axis_name="s",
                                  num_cores=NC, num_subcores=NS)
    out_type = [
        jax.ShapeDtypeStruct((HEADS, N_NODES, OUT), _f32),
        jax.ShapeDtypeStruct((HEADS, N_HE, OUT), _f32),
    ]
    scratch = [
        pltpu.VMEM((2, EPB, OUT), _f32),
        pltpu.VMEM((2, EPB), jnp.int32),
        pltpu.VMEM((2, EPB), jnp.int32),
        pltpu.VMEM((2, EPB), _f32),
        pltpu.VMEM((OCH,), _f32),
        pltpu.VMEM((OUT,), _f32),
        pltpu.VMEM_SHARED((N_HE, OUT), _f32),
        pltpu.SemaphoreType.DMA,
        pltpu.SemaphoreType.DMA,
        pltpu.SemaphoreType.DMA,
    ]

    def body(src1, dst1, xh_hm, alpha, alpha1, drec, bias_hm,
             h_o, oute_o,
             rowbuf, sidx, didx, abuf, dbuf, biasv, acc,
             semG, semS, semL):
        c = lax.axis_index("c")
        s = lax.axis_index("s")

        def zrows(h):
            def zr(r, _):
                rowbuf[h, r, pl.ds(0, LN)] = jnp.zeros((LN,), _f32)
                rowbuf[h, r, pl.ds(LN, LN)] = jnp.zeros((LN,), _f32)
                return _
            lax.fori_loop(0, EPB, zr, None)

        zrows(0)
        zrows(1)
        st = s * SEG_FULL

        def zseg(total):
            o = 0
            while o < total:
                nsz = min(EPB, total - o)
                if nsz == EPB:
                    pltpu.sync_copy(rowbuf.at[0], acc.at[pl.ds(st + o, nsz)])
                else:
                    pltpu.sync_copy(rowbuf.at[0, pl.ds(0, nsz)],
                                    acc.at[pl.ds(st + o, nsz)])
                o += nsz

        @pl.when(s < NS - 1)
        def _():
            zseg(SEG_FULL)

        @pl.when(s == NS - 1)
        def _():
            zseg(SEG_LAST)
        plsc.subcore_barrier()

        def propagate(table_hbm, gidx, scatteridx, wsrc, zero_set1):
            tbl = table_hbm.at[c]

            def ebase(ch):
                return (s * ROWS_PER_TILE + ch * CHB) * BW

            def g_of(h):
                return sidx.at[h] if gidx == "src" else didx.at[h]

            def sc_of(h):
                return sidx.at[h] if scatteridx == "src" else didx.at[h]

            def issue_loads(ch, h):
                eb = ebase(ch)
                pltpu.async_copy(src1.at[pl.ds(eb, EPB)], sidx.at[h], semL)
                pltpu.async_copy(dst1.at[pl.ds(eb, EPB)], didx.at[h], semL)
                pltpu.async_copy(wsrc.at[c, pl.ds(eb, EPB)], abuf.at[h], semL)

            def wait_loads(h):
                pltpu.make_async_copy(src1.at[pl.ds(0, EPB)], sidx.at[h],
                                      semL).wait()
                pltpu.make_async_copy(dst1.at[pl.ds(0, EPB)], didx.at[h],
                                      semL).wait()
                pltpu.make_async_copy(wsrc.at[c, pl.ds(0, EPB)], abuf.at[h],
                                      semL).wait()

            def issue_gather(h):
                pltpu.async_copy(tbl.at[g_of(h)], rowbuf.at[h], semG)

            def wait_gather(h):
                pltpu.make_async_copy(tbl.at[g_of(h)], rowbuf.at[h],
                                      semG).wait()

            def issue_scatter(h):
                pltpu.async_copy(rowbuf.at[h], acc.at[sc_of(h)], semS,
                                 add=True)

            def wait_scatter(h):
                pltpu.make_async_copy(rowbuf.at[h], acc.at[sc_of(h)],
                                      semS).wait()

            def compute(h):
                @plsc.parallel_loop(0, EPB // LN, 1, unroll=2)
                def _(q):
                    for i in range(LN):
                        e = q * LN + i
                        w = plsc.load_gather(
                            abuf.at[h], [jnp.full((LN,), e, jnp.int32)])
                        rowbuf[h, e, pl.ds(0, LN)] = \
                            rowbuf[h, e, pl.ds(0, LN)] * w
                        rowbuf[h, e, pl.ds(LN, LN)] = \
                            rowbuf[h, e, pl.ds(LN, LN)] * w

            if zero_set1:
                zrows(1)
            pltpu.sync_copy(src1.at[pl.ds(ebase(0), EPB)], sidx.at[1])
            pltpu.sync_copy(dst1.at[pl.ds(ebase(0), EPB)], didx.at[1])
            issue_scatter(1)
            issue_loads(0, 0)
            wait_loads(0)
            issue_gather(0)

            def pair(i, _):
                for h in (0, 1):
                    ch = 2 * i + h
                    wait_scatter(1 - h)
                    issue_loads(ch + 1, 1 - h)
                    wait_gather(h)
                    wait_loads(1 - h)
                    issue_gather(1 - h)
                    compute(h)
                    issue_scatter(h)
                return _
            lax.fori_loop(0, (NCHB - 1) // 2, pair, None)

            wait_scatter(1)
            wait_gather(0)
            compute(0)
            issue_scatter(0)
            wait_scatter(0)

        propagate(xh_hm, "src", "dst", alpha1, zero_set1=False)
        plsc.subcore_barrier()

        zrows(1)
        r0t = s * OTILE

        def wout(nchunks):
            for k in range(nchunks):
                r0 = r0t + k * OCH
                pltpu.sync_copy(acc.at[pl.ds(r0, OCH)], rowbuf.at[0])
                pltpu.sync_copy(rowbuf.at[0], oute_o.at[c, pl.ds(r0, OCH)])
                pltpu.sync_copy(rowbuf.at[1], acc.at[pl.ds(r0, OCH)])

        @pl.when(s < NS - 1)
        def _():
            wout(OTILE // OCH)

        @pl.when(s == NS - 1)
        def _():
            wout(OLAST // OCH)
        plsc.subcore_barrier()

        propagate(oute_o, "dst", "src", alpha, zero_set1=True)
        plsc.subcore_barrier()

        pltpu.sync_copy(bias_hm.at[c], biasv)
        blo = biasv[pl.ds(0, LN)]
        bhi = biasv[pl.ds(LN, LN)]

        def fin(nchunks):
            for k in range(nchunks):
                r0 = r0t + k * OCH
                pltpu.sync_copy(acc.at[pl.ds(r0, OCH)], rowbuf.at[0])
                pltpu.sync_copy(drec.at[pl.ds(r0, OCH)], dbuf)

                def frow(r, _):
                    dv = plsc.load_gather(dbuf,
                                          [jnp.full((LN,), r, jnp.int32)])
                    lo = rowbuf[0, r, pl.ds(0, LN)] * dv + blo
                    hi = rowbuf[0, r, pl.ds(LN, LN)] * dv + bhi
                    if first_layer:
                        lo = jnp.maximum(lo, 0.0)
                        hi = jnp.maximum(hi, 0.0)
                    rowbuf[0, r, pl.ds(0, LN)] = lo
                    rowbuf[0, r, pl.ds(LN, LN)] = hi
                    return _
                lax.fori_loop(0, OCH, frow, None)
                pltpu.sync_copy(rowbuf.at[0], h_o.at[c, pl.ds(r0, OCH)])

        @pl.when(s < NS - 1)
        def _():
            fin(OTILE // OCH)

        @pl.when(s == NS - 1)
        def _():
            fin(OLAST // OCH)

    return pl.kernel(body, out_type=out_type, mesh=mesh,
                     scratch_types=scratch,
                     compiler_params=pltpu.CompilerParams(
                         use_tc_tiling_on_sc=False,
                         needs_layout_passes=False))


def _att_mats(att):
    a = att[0]
    Ai = jnp.zeros((D_IN, HEADS), _f32)
    Aj = jnp.zeros((D_IN, HEADS), _f32)
    for h in range(HEADS):
        Ai = Ai.at[h * OUT:(h + 1) * OUT, h].set(a[h, :OUT])
        Aj = Aj.at[h * OUT:(h + 1) * OUT, h].set(a[h, OUT:])
    return Ai, Aj


@jax.jit
def kernel(x, hyperedges, hyperedge_attr, W0, att0, b0, W1, att1, b1):
    src1 = hyperedges[0]
    dst1 = hyperedges[1]

    sc_a0 = _make_sc_a(True)
    sc_a1 = _make_sc_a(False)
    sc_b0 = _make_sc_b(True)
    sc_b1 = _make_sc_b(False)

    def layer(F, W, att, b, sc_a, sc_b, extra):
        Ai, Aj = _att_mats(att)
        xh_hm, ai, aj, m4 = _tc_embed(F, hyperedge_attr, W, Ai, Aj)
        ai_hm = ai.T
        aj_hm = aj.T
        m = m4[:2] + m4[2:]
        m = jnp.maximum(m, 0.2 * m)
        mh = jnp.tile(m[:, None], (1, LN))
        if extra is None:
            _ex, alpha, alpha1, drec, cntdst = sc_a(
                src1, dst1, ai_hm, aj_hm, mh)
        else:
            drec, cntdst = extra
            _ex, alpha, alpha1 = sc_a(src1, dst1, ai_hm, aj_hm, mh, cntdst)
        bias_hm = b.reshape(HEADS, OUT)
        h_hm, _oute = sc_b(src1, dst1, xh_hm, alpha, alpha1, drec, bias_hm)
        return h_hm, (drec, cntdst)

    h_hm, dd = layer(x, W0, att0, b0, sc_a0, sc_b0, None)
    h64 = h_hm.transpose(1, 0, 2).reshape(N_NODES, HEADS * OUT)
    out_hm, _ = layer(h64, W1, att1, b1, sc_a1, sc_b1, dd)
    return out_hm.transpose(1, 0, 2).reshape(N_NODES, HEADS * OUT)


# speedup vs baseline: 195.5385x; 1.1849x over previous
import functools

import jax
import jax.numpy as jnp
from jax import lax
from jax.experimental import pallas as pl
from jax.experimental.pallas import tpu as pltpu
from jax.experimental.pallas import tpu_sc as plsc

N_NODES = 50000
N_HE = 50000
NNZ = 800000
HEADS = 2
OUT = 32
D_IN = 64

NC = 2
NS = 16
LN = 16

BW = 80
IDX_ROWS = NNZ // BW
ROWS_PER_TILE = IDX_ROWS // NS
CHUNK_ROWS = 25
EDGES_PER_CHUNK = CHUNK_ROWS * BW
NCHUNKS = ROWS_PER_TILE // CHUNK_ROWS

SEG_FULL = 3136
SEG_LAST = N_HE - 15 * SEG_FULL
OTILE = 3200
OLAST = N_NODES - 15 * OTILE
OCH = 400
CHB = 5
EPB = CHB * BW
NCHB = ROWS_PER_TILE // CHB

_EPS = 1e-16
_f32 = jnp.float32


def _seg_range(s):
    return s * SEG_FULL


def _tc_embed(F, he_attr, W, Ai, Aj):
    n = F.shape[0]
    blk = 1000
    grid = n // blk

    def body(f_ref, he_ref, w_ref, ai_w_ref, aj_w_ref,
             xh_ref, ai_ref, aj_ref, m4_ref, msc):
        b = pl.program_id(0)
        xh = jnp.dot(f_ref[...], w_ref[...], preferred_element_type=_f32)
        heh = jnp.dot(he_ref[...], w_ref[...], preferred_element_type=_f32)
        ai = jnp.dot(xh, ai_w_ref[...], preferred_element_type=_f32)
        aj = jnp.dot(heh, aj_w_ref[...], preferred_element_type=_f32)
        xh_ref[0] = xh[:, :OUT]
        xh_ref[1] = xh[:, OUT:]
        ai_ref[...] = ai
        aj_ref[...] = aj
        mi0 = jnp.max(ai[:, 0])
        mi1 = jnp.max(ai[:, 1])
        mj0 = jnp.max(aj[:, 0])
        mj1 = jnp.max(aj[:, 1])

        @pl.when(b == 0)
        def _():
            msc[0], msc[1], msc[2], msc[3] = mi0, mi1, mj0, mj1

        @pl.when(b > 0)
        def _():
            msc[0] = jnp.maximum(msc[0], mi0)
            msc[1] = jnp.maximum(msc[1], mi1)
            msc[2] = jnp.maximum(msc[2], mj0)
            msc[3] = jnp.maximum(msc[3], mj1)

        @pl.when(b == pl.num_programs(0) - 1)
        def _():
            m4_ref[0], m4_ref[1] = msc[0], msc[1]
            m4_ref[2], m4_ref[3] = msc[2], msc[3]

    return pl.pallas_call(
        body,
        grid=(grid,),
        in_specs=[
            pl.BlockSpec((blk, D_IN), lambda b: (b, 0)),
            pl.BlockSpec((blk, D_IN), lambda b: (b, 0)),
            pl.BlockSpec((D_IN, D_IN), lambda b: (0, 0)),
            pl.BlockSpec((D_IN, HEADS), lambda b: (0, 0)),
            pl.BlockSpec((D_IN, HEADS), lambda b: (0, 0)),
        ],
        out_specs=[
            pl.BlockSpec((HEADS, blk, OUT), lambda b: (0, b, 0)),
            pl.BlockSpec((blk, HEADS), lambda b: (b, 0)),
            pl.BlockSpec((blk, HEADS), lambda b: (b, 0)),
            pl.BlockSpec(memory_space=pltpu.SMEM),
        ],
        out_shape=[
            jax.ShapeDtypeStruct((HEADS, n, OUT), _f32),
            jax.ShapeDtypeStruct((n, HEADS), _f32),
            jax.ShapeDtypeStruct((he_attr.shape[0], HEADS), _f32),
            jax.ShapeDtypeStruct((4,), _f32),
        ],
        scratch_shapes=[pltpu.SMEM((4,), _f32)],
    )(F, he_attr, W, Ai, Aj)


def _zero_1d(buf, nwords):
    def zb(k, _):
        buf[pl.ds(k * LN, LN)] = jnp.zeros((LN,), _f32)
        return _
    lax.fori_loop(0, nwords // LN, zb, None)


def _make_sc_a(first_layer):
    mesh = plsc.VectorSubcoreMesh(core_axis_name="c", subcore_axis_name="s",
                                  num_cores=NC, num_subcores=NS)

    out_type = [
        jax.ShapeDtypeStruct((HEADS, NNZ), _f32),
        jax.ShapeDtypeStruct((HEADS, NNZ), _f32),
        jax.ShapeDtypeStruct((HEADS, NNZ), _f32),
    ]
    if first_layer:
        out_type += [
            jax.ShapeDtypeStruct((N_NODES,), _f32),
            jax.ShapeDtypeStruct((N_HE,), _f32),
        ]

    scratch = dict(
        tbl_a=pltpu.VMEM((N_NODES,), _f32),
        tbl_b=pltpu.VMEM((N_HE,), _f32),
        sidx=pltpu.VMEM((EDGES_PER_CHUNK,), jnp.int32),
        didx=pltpu.VMEM((EDGES_PER_CHUNK,), jnp.int32),
        ebuf=pltpu.VMEM((EDGES_PER_CHUNK,), _f32),
        a1buf=pltpu.VMEM((EDGES_PER_CHUNK,), _f32),
        ones=pltpu.VMEM((EDGES_PER_CHUNK,), _f32),
        mbuf=pltpu.VMEM((LN,), _f32),
        d1=pltpu.VMEM((SEG_FULL,), _f32),
        d2=pltpu.VMEM((SEG_FULL,), _f32),
        d3=pltpu.VMEM((SEG_FULL,), _f32),
        den_sp=pltpu.VMEM_SHARED((N_HE,), _f32),
        cdst_sp=pltpu.VMEM_SHARED((N_HE,), _f32),
        csrc_sp=pltpu.VMEM_SHARED((N_NODES,), _f32),
    )

    def body(src1, dst1, ai_hm, aj_hm, mh, cntdst_in,
             ex_o, alpha_o, alpha1_o, drec_o, cntdst_o,
             tbl_a, tbl_b, sidx, didx, ebuf, a1buf, ones, mbuf,
             d1, d2, d3, den_sp, cdst_sp, csrc_sp):
        c = lax.axis_index("c")
        s = lax.axis_index("s")

        def ffill(buf, n, val):
            def zb(k, _):
                buf[pl.ds(k * LN, LN)] = jnp.full((LN,), val, _f32)
                return _
            lax.fori_loop(0, n // LN, zb, None)
        ffill(ones, EDGES_PER_CHUNK, 1.0)
        ffill(d1, SEG_FULL, 0.0)
        st = s * SEG_FULL

        for sp in ([den_sp, cdst_sp, csrc_sp] if first_layer else [den_sp]):
            @pl.when(s < NS - 1)
            def _(sp=sp):
                pltpu.sync_copy(d1.at[pl.ds(0, SEG_FULL)],
                                sp.at[pl.ds(st, SEG_FULL)])

            @pl.when(s == NS - 1)
            def _(sp=sp):
                pltpu.sync_copy(d1.at[pl.ds(0, SEG_LAST)],
                                sp.at[pl.ds(st, SEG_LAST)])
        plsc.subcore_barrier()

        pltpu.sync_copy(ai_hm.at[c], tbl_a)
        pltpu.sync_copy(aj_hm.at[c], tbl_b)
        pltpu.sync_copy(mh.at[c], mbuf)
        mvec = mbuf[...]

        def p1_chunk(ch, _):
            eb = (s * ROWS_PER_TILE + ch * CHUNK_ROWS) * BW
            pltpu.sync_copy(src1.at[pl.ds(eb, EDGES_PER_CHUNK)], sidx)
            pltpu.sync_copy(dst1.at[pl.ds(eb, EDGES_PER_CHUNK)], didx)

            @plsc.parallel_loop(0, EDGES_PER_CHUNK // LN, 1, unroll=4)
            def _(k):
                sl = pl.ds(k * LN, LN)
                av = plsc.load_gather(tbl_a, [sidx[sl]])
                bv = plsc.load_gather(tbl_b, [didx[sl]])
                logit = av + bv
                logit = jnp.maximum(logit, 0.2 * logit)
                ebuf[sl] = jnp.exp(logit - mvec)

            pltpu.sync_copy(ebuf, ex_o.at[c, pl.ds(eb, EDGES_PER_CHUNK)])
            pltpu.sync_copy(ebuf, den_sp.at[didx], add=True)
            if first_layer:
                pltpu.sync_copy(ones, cdst_sp.at[didx], add=True)
                pltpu.sync_copy(ones, csrc_sp.at[sidx], add=True)
            return _
        lax.fori_loop(0, NCHUNKS, p1_chunk, None)
        plsc.subcore_barrier()

        def p15(size):
            pltpu.sync_copy(den_sp.at[pl.ds(st, size)], d1.at[pl.ds(0, size)])
            if first_layer:
                pltpu.sync_copy(cdst_sp.at[pl.ds(st, size)],
                                d2.at[pl.ds(0, size)])
                pltpu.sync_copy(csrc_sp.at[pl.ds(st, size)],
                                d3.at[pl.ds(0, size)])

                @pl.when(c == 1)
                def _():
                    pltpu.sync_copy(d2.at[pl.ds(0, size)],
                                    cntdst_o.at[pl.ds(st, size)])
            else:
                pltpu.sync_copy(cntdst_in.at[pl.ds(st, size)],
                                d2.at[pl.ds(0, size)])

            def vec(k, _):
                sl = pl.ds(k * LN, LN)
                de = d1[sl] + _EPS
                d1[sl] = de
                d2[sl] = de * d2[sl]
                if first_layer:
                    cs = d3[sl]
                    d3[sl] = jnp.where(cs > 0.0, 1.0 / cs, 0.0)
                return _
            lax.fori_loop(0, size // LN, vec, None)

            pltpu.sync_copy(d1.at[pl.ds(0, size)], den_sp.at[pl.ds(st, size)])
            pltpu.sync_copy(d2.at[pl.ds(0, size)], cdst_sp.at[pl.ds(st, size)])
            if first_layer:
                @pl.when(c == 0)
                def _():
                    pltpu.sync_copy(d3.at[pl.ds(0, size)],
                                    drec_o.at[pl.ds(st, size)])

        @pl.when(s < NS - 1)
        def _():
            p15(SEG_FULL)

        @pl.when(s == NS - 1)
        def _():
            p15(SEG_LAST)
        plsc.subcore_barrier()

        pltpu.sync_copy(den_sp, tbl_a)
        pltpu.sync_copy(cdst_sp, tbl_b)

        def p2_chunk(ch, _):
            eb = (s * ROWS_PER_TILE + ch * CHUNK_ROWS) * BW
            pltpu.sync_copy(dst1.at[pl.ds(eb, EDGES_PER_CHUNK)], didx)
            pltpu.sync_copy(ex_o.at[c, pl.ds(eb, EDGES_PER_CHUNK)], ebuf)

            @plsc.parallel_loop(0, EDGES_PER_CHUNK // LN, 1, unroll=4)
            def _(k):
                sl = pl.ds(k * LN, LN)
                jv = didx[sl]
                exv = ebuf[sl]
                d0 = plsc.load_gather(tbl_a, [jv])
                dd1 = plsc.load_gather(tbl_b, [jv])
                ebuf[sl] = exv / d0
                a1buf[sl] = exv / dd1
            pltpu.sync_copy(ebuf, alpha_o.at[c, pl.ds(eb, EDGES_PER_CHUNK)])
            pltpu.sync_copy(a1buf, alpha1_o.at[c, pl.ds(eb, EDGES_PER_CHUNK)])
            return _
        lax.fori_loop(0, NCHUNKS, p2_chunk, None)

    cparams = pltpu.CompilerParams(use_tc_tiling_on_sc=False,
                                   needs_layout_passes=False)
    if first_layer:
        def call(src1, dst1, ai_hm, aj_hm, mh):
            def body0(src1, dst1, ai_hm, aj_hm, mh, *rest):
                return body(src1, dst1, ai_hm, aj_hm, mh, None, *rest)
            fn = pl.kernel(body0, out_type=out_type, mesh=mesh,
                           scratch_types=list(scratch.values()),
                           compiler_params=cparams)
            return fn(src1, dst1, ai_hm, aj_hm, mh)
        return call
    else:
        def call(src1, dst1, ai_hm, aj_hm, mh, cntdst):
            def body1(src1, dst1, ai_hm, aj_hm, mh, cntdst_in,
                      ex_o, alpha_o, alpha1_o, *rest):
                return body(src1, dst1, ai_hm, aj_hm, mh, cntdst_in,
                            ex_o, alpha_o, alpha1_o, None, None, *rest)
            fn = pl.kernel(body1, out_type=out_type, mesh=mesh,
                           scratch_types=list(scratch.values()),
                           compiler_params=cparams)
            return fn(src1, dst1, ai_hm, aj_hm, mh, cntdst)
        return call


def _make_sc_b(first_layer):
    mesh = plsc.VectorSubcoreMesh(core_axis_name="c", subcore_axis_name="s",
                                  num_cores=NC, num_subcores=NS)
    out_type = [
        jax.ShapeDtypeStruct((HEADS, N_NODES, OUT), _f32),
        jax.ShapeDtypeStruct((HEADS, N_HE, OUT), _f32),
    ]
    scratch = [
        pltpu.VMEM((2, EPB, OUT), _f32),
        pltpu.VMEM((2, EPB), jnp.int32),
        pltpu.VMEM((2, EPB), jnp.int32),
        pltpu.VMEM((2, EPB), _f32),
        pltpu.VMEM((OCH,), _f32),
        pltpu.VMEM((OUT,), _f32),
        pltpu.VMEM_SHARED((N_HE, OUT), _f32),
        pltpu.SemaphoreType.DMA,
        pltpu.SemaphoreType.DMA,
        pltpu.SemaphoreType.DMA,
    ]

    def body(src1, dst1, xh_hm, alpha, alpha1, drec, bias_hm,
             h_o, oute_o,
             rowbuf, sidx, didx, abuf, dbuf, biasv, acc,
             semG, semS, semL):
        c = lax.axis_index("c")
        s = lax.axis_index("s")

        def zrows(h):
            def zr(r, _):
                rowbuf[h, r, pl.ds(0, LN)] = jnp.zeros((LN,), _f32)
                rowbuf[h, r, pl.ds(LN, LN)] = jnp.zeros((LN,), _f32)
                return _
            lax.fori_loop(0, EPB, zr, None)

        zrows(0)
        zrows(1)
        st = s * SEG_FULL

        def zseg(total):
            o = 0
            while o < total:
                nsz = min(EPB, total - o)
                if nsz == EPB:
                    pltpu.sync_copy(rowbuf.at[0], acc.at[pl.ds(st + o, nsz)])
                else:
                    pltpu.sync_copy(rowbuf.at[0, pl.ds(0, nsz)],
                                    acc.at[pl.ds(st + o, nsz)])
                o += nsz

        @pl.when(s < NS - 1)
        def _():
            zseg(SEG_FULL)

        @pl.when(s == NS - 1)
        def _():
            zseg(SEG_LAST)
        plsc.subcore_barrier()

        def propagate(table_hbm, gidx, scatteridx, wsrc, zero_set1):
            tbl = table_hbm.at[c]

            def ebase(ch):
                return (s * ROWS_PER_TILE + ch * CHB) * BW

            def g_of(h):
                return sidx.at[h] if gidx == "src" else didx.at[h]

            def sc_of(h):
                return sidx.at[h] if scatteridx == "src" else didx.at[h]

            def issue_loads(ch, h):
                eb = ebase(ch)
                pltpu.async_copy(src1.at[pl.ds(eb, EPB)], sidx.at[h], semL)
                pltpu.async_copy(dst1.at[pl.ds(eb, EPB)], didx.at[h], semL)
                pltpu.async_copy(wsrc.at[c, pl.ds(eb, EPB)], abuf.at[h], semL)

            def wait_loads(h):
                pltpu.make_async_copy(src1.at[pl.ds(0, EPB)], sidx.at[h],
                                      semL).wait()
                pltpu.make_async_copy(dst1.at[pl.ds(0, EPB)], didx.at[h],
                                      semL).wait()
                pltpu.make_async_copy(wsrc.at[c, pl.ds(0, EPB)], abuf.at[h],
                                      semL).wait()

            def issue_gather(h):
                pltpu.async_copy(tbl.at[g_of(h)], rowbuf.at[h], semG)

            def wait_gather(h):
                pltpu.make_async_copy(tbl.at[g_of(h)], rowbuf.at[h],
                                      semG).wait()

            def issue_scatter(h):
                pltpu.async_copy(rowbuf.at[h], acc.at[sc_of(h)], semS,
                                 add=True)

            def wait_scatter(h):
                pltpu.make_async_copy(rowbuf.at[h], acc.at[sc_of(h)],
                                      semS).wait()

            def compute(h):
                @plsc.parallel_loop(0, EPB // LN, 1, unroll=4)
                def _(q):
                    for i in range(LN):
                        e = q * LN + i
                        w = plsc.load_gather(
                            abuf.at[h], [jnp.full((LN,), e, jnp.int32)])
                        rowbuf[h, e, pl.ds(0, LN)] = \
                            rowbuf[h, e, pl.ds(0, LN)] * w
                        rowbuf[h, e, pl.ds(LN, LN)] = \
                            rowbuf[h, e, pl.ds(LN, LN)] * w

            if zero_set1:
                zrows(1)
            pltpu.sync_copy(src1.at[pl.ds(ebase(0), EPB)], sidx.at[1])
            pltpu.sync_copy(dst1.at[pl.ds(ebase(0), EPB)], didx.at[1])
            issue_scatter(1)
            issue_loads(0, 0)
            wait_loads(0)
            issue_gather(0)

            def pair(i, _):
                for h in (0, 1):
                    ch = 2 * i + h
                    wait_scatter(1 - h)
                    issue_loads(ch + 1, 1 - h)
                    wait_gather(h)
                    wait_loads(1 - h)
                    issue_gather(1 - h)
                    compute(h)
                    issue_scatter(h)
                return _
            lax.fori_loop(0, (NCHB - 1) // 2, pair, None)

            wait_scatter(1)
            wait_gather(0)
            compute(0)
            issue_scatter(0)
            wait_scatter(0)

        propagate(xh_hm, "src", "dst", alpha1, zero_set1=False)
        plsc.subcore_barrier()

        zrows(1)
        r0t = s * OTILE

        def wout(nchunks):
            for k in range(nchunks):
                r0 = r0t + k * OCH
                pltpu.sync_copy(acc.at[pl.ds(r0, OCH)], rowbuf.at[0])
                pltpu.sync_copy(rowbuf.at[0], oute_o.at[c, pl.ds(r0, OCH)])
                pltpu.sync_copy(rowbuf.at[1], acc.at[pl.ds(r0, OCH)])

        @pl.when(s < NS - 1)
        def _():
            wout(OTILE // OCH)

        @pl.when(s == NS - 1)
        def _():
            wout(OLAST // OCH)
        plsc.subcore_barrier()

        propagate(oute_o, "dst", "src", alpha, zero_set1=True)
        plsc.subcore_barrier()

        pltpu.sync_copy(bias_hm.at[c], biasv)
        blo = biasv[pl.ds(0, LN)]
        bhi = biasv[pl.ds(LN, LN)]

        def fin(nchunks):
            for k in range(nchunks):
                r0 = r0t + k * OCH
                pltpu.sync_copy(acc.at[pl.ds(r0, OCH)], rowbuf.at[0])
                pltpu.sync_copy(drec.at[pl.ds(r0, OCH)], dbuf)

                @plsc.parallel_loop(0, OCH, 1, unroll=4)
                def _(r):
                    dv = plsc.load_gather(dbuf,
                                          [jnp.full((LN,), r, jnp.int32)])
                    lo = rowbuf[0, r, pl.ds(0, LN)] * dv + blo
                    hi = rowbuf[0, r, pl.ds(LN, LN)] * dv + bhi
                    if first_layer:
                        lo = jnp.maximum(lo, 0.0)
                        hi = jnp.maximum(hi, 0.0)
                    rowbuf[0, r, pl.ds(0, LN)] = lo
                    rowbuf[0, r, pl.ds(LN, LN)] = hi
                pltpu.sync_copy(rowbuf.at[0], h_o.at[c, pl.ds(r0, OCH)])

        @pl.when(s < NS - 1)
        def _():
            fin(OTILE // OCH)

        @pl.when(s == NS - 1)
        def _():
            fin(OLAST // OCH)

    return pl.kernel(body, out_type=out_type, mesh=mesh,
                     scratch_types=scratch,
                     compiler_params=pltpu.CompilerParams(
                         use_tc_tiling_on_sc=False,
                         needs_layout_passes=False))


def _att_mats(att):
    a = att[0]
    Ai = jnp.zeros((D_IN, HEADS), _f32)
    Aj = jnp.zeros((D_IN, HEADS), _f32)
    for h in range(HEADS):
        Ai = Ai.at[h * OUT:(h + 1) * OUT, h].set(a[h, :OUT])
        Aj = Aj.at[h * OUT:(h + 1) * OUT, h].set(a[h, OUT:])
    return Ai, Aj


@jax.jit
def kernel(x, hyperedges, hyperedge_attr, W0, att0, b0, W1, att1, b1):
    src1 = hyperedges[0]
    dst1 = hyperedges[1]

    sc_a0 = _make_sc_a(True)
    sc_a1 = _make_sc_a(False)
    sc_b0 = _make_sc_b(True)
    sc_b1 = _make_sc_b(False)

    def layer(F, W, att, b, sc_a, sc_b, extra):
        Ai, Aj = _att_mats(att)
        xh_hm, ai, aj, m4 = _tc_embed(F, hyperedge_attr, W, Ai, Aj)
        ai_hm = ai.T
        aj_hm = aj.T
        m = m4[:2] + m4[2:]
        m = jnp.maximum(m, 0.2 * m)
        mh = jnp.tile(m[:, None], (1, LN))
        if extra is None:
            _ex, alpha, alpha1, drec, cntdst = sc_a(
                src1, dst1, ai_hm, aj_hm, mh)
        else:
            drec, cntdst = extra
            _ex, alpha, alpha1 = sc_a(src1, dst1, ai_hm, aj_hm, mh, cntdst)
        bias_hm = b.reshape(HEADS, OUT)
        h_hm, _oute = sc_b(src1, dst1, xh_hm, alpha, alpha1, drec, bias_hm)
        return h_hm, (drec, cntdst)

    h_hm, dd = layer(x, W0, att0, b0, sc_a0, sc_b0, None)
    h64 = h_hm.transpose(1, 0, 2).reshape(N_NODES, HEADS * OUT)
    out_hm, _ = layer(h64, W1, att1, b1, sc_a1, sc_b1, dd)
    return out_hm.transpose(1, 0, 2).reshape(N_NODES, HEADS * OUT)


# speedup vs baseline: 199.2749x; 1.0191x over previous
import functools

import jax
import jax.numpy as jnp
from jax import lax
from jax.experimental import pallas as pl
from jax.experimental.pallas import tpu as pltpu
from jax.experimental.pallas import tpu_sc as plsc

N_NODES = 50000
N_HE = 50000
NNZ = 800000
HEADS = 2
OUT = 32
D_IN = 64

NC = 2
NS = 16
LN = 16

BW = 80
IDX_ROWS = NNZ // BW
ROWS_PER_TILE = IDX_ROWS // NS
CHUNK_ROWS = 25
EDGES_PER_CHUNK = CHUNK_ROWS * BW
NCHUNKS = ROWS_PER_TILE // CHUNK_ROWS

SEG_FULL = 3136
SEG_LAST = N_HE - 15 * SEG_FULL
OTILE = 3200
OLAST = N_NODES - 15 * OTILE
OCH = 400
CHB = 5
EPB = CHB * BW
NCHB = ROWS_PER_TILE // CHB

_EPS = 1e-16
_f32 = jnp.float32


def _seg_range(s):
    return s * SEG_FULL


def _tc_embed(F, he_attr, W, Ai, Aj):
    n = F.shape[0]
    blk = 1000
    grid = n // blk

    def body(f_ref, he_ref, w_ref, ai_w_ref, aj_w_ref,
             xh_ref, ai_ref, aj_ref, m4_ref, msc):
        b = pl.program_id(0)
        xh = jnp.dot(f_ref[...], w_ref[...], preferred_element_type=_f32)
        heh = jnp.dot(he_ref[...], w_ref[...], preferred_element_type=_f32)
        ai = jnp.dot(xh, ai_w_ref[...], preferred_element_type=_f32)
        aj = jnp.dot(heh, aj_w_ref[...], preferred_element_type=_f32)
        xh_ref[0] = xh[:, :OUT]
        xh_ref[1] = xh[:, OUT:]
        ai_ref[...] = ai
        aj_ref[...] = aj
        mi0 = jnp.max(ai[:, 0])
        mi1 = jnp.max(ai[:, 1])
        mj0 = jnp.max(aj[:, 0])
        mj1 = jnp.max(aj[:, 1])

        @pl.when(b == 0)
        def _():
            msc[0], msc[1], msc[2], msc[3] = mi0, mi1, mj0, mj1

        @pl.when(b > 0)
        def _():
            msc[0] = jnp.maximum(msc[0], mi0)
            msc[1] = jnp.maximum(msc[1], mi1)
            msc[2] = jnp.maximum(msc[2], mj0)
            msc[3] = jnp.maximum(msc[3], mj1)

        @pl.when(b == pl.num_programs(0) - 1)
        def _():
            m4_ref[0], m4_ref[1] = msc[0], msc[1]
            m4_ref[2], m4_ref[3] = msc[2], msc[3]

    return pl.pallas_call(
        body,
        grid=(grid,),
        in_specs=[
            pl.BlockSpec((blk, D_IN), lambda b: (b, 0)),
            pl.BlockSpec((blk, D_IN), lambda b: (b, 0)),
            pl.BlockSpec((D_IN, D_IN), lambda b: (0, 0)),
            pl.BlockSpec((D_IN, HEADS), lambda b: (0, 0)),
            pl.BlockSpec((D_IN, HEADS), lambda b: (0, 0)),
        ],
        out_specs=[
            pl.BlockSpec((HEADS, blk, OUT), lambda b: (0, b, 0)),
            pl.BlockSpec((blk, HEADS), lambda b: (b, 0)),
            pl.BlockSpec((blk, HEADS), lambda b: (b, 0)),
            pl.BlockSpec(memory_space=pltpu.SMEM),
        ],
        out_shape=[
            jax.ShapeDtypeStruct((HEADS, n, OUT), _f32),
            jax.ShapeDtypeStruct((n, HEADS), _f32),
            jax.ShapeDtypeStruct((he_attr.shape[0], HEADS), _f32),
            jax.ShapeDtypeStruct((4,), _f32),
        ],
        scratch_shapes=[pltpu.SMEM((4,), _f32)],
    )(F, he_attr, W, Ai, Aj)


def _zero_1d(buf, nwords):
    def zb(k, _):
        buf[pl.ds(k * LN, LN)] = jnp.zeros((LN,), _f32)
        return _
    lax.fori_loop(0, nwords // LN, zb, None)


def _make_sc_a(first_layer):
    mesh = plsc.VectorSubcoreMesh(core_axis_name="c", subcore_axis_name="s",
                                  num_cores=NC, num_subcores=NS)

    out_type = [
        jax.ShapeDtypeStruct((HEADS, NNZ), _f32),
        jax.ShapeDtypeStruct((HEADS, NNZ), _f32),
        jax.ShapeDtypeStruct((HEADS, NNZ), _f32),
    ]
    if first_layer:
        out_type += [
            jax.ShapeDtypeStruct((N_NODES,), _f32),
            jax.ShapeDtypeStruct((N_HE,), _f32),
        ]

    scratch = dict(
        tbl_a=pltpu.VMEM((N_NODES,), _f32),
        tbl_b=pltpu.VMEM((N_HE,), _f32),
        sidx=pltpu.VMEM((EDGES_PER_CHUNK,), jnp.int32),
        didx=pltpu.VMEM((EDGES_PER_CHUNK,), jnp.int32),
        ebuf=pltpu.VMEM((EDGES_PER_CHUNK,), _f32),
        a1buf=pltpu.VMEM((EDGES_PER_CHUNK,), _f32),
        ones=pltpu.VMEM((EDGES_PER_CHUNK,), _f32),
        mbuf=pltpu.VMEM((LN,), _f32),
        d1=pltpu.VMEM((SEG_FULL,), _f32),
        d2=pltpu.VMEM((SEG_FULL,), _f32),
        d3=pltpu.VMEM((SEG_FULL,), _f32),
        den_sp=pltpu.VMEM_SHARED((N_HE,), _f32),
        cdst_sp=pltpu.VMEM_SHARED((N_HE,), _f32),
        csrc_sp=pltpu.VMEM_SHARED((N_NODES,), _f32),
    )

    def body(src1, dst1, ai_hm, aj_hm, mh, cntdst_in,
             ex_o, alpha_o, alpha1_o, drec_o, cntdst_o,
             tbl_a, tbl_b, sidx, didx, ebuf, a1buf, ones, mbuf,
             d1, d2, d3, den_sp, cdst_sp, csrc_sp):
        c = lax.axis_index("c")
        s = lax.axis_index("s")

        def ffill(buf, n, val):
            def zb(k, _):
                buf[pl.ds(k * LN, LN)] = jnp.full((LN,), val, _f32)
                return _
            lax.fori_loop(0, n // LN, zb, None)
        ffill(ones, EDGES_PER_CHUNK, 1.0)
        ffill(d1, SEG_FULL, 0.0)
        st = s * SEG_FULL

        for sp in ([den_sp, cdst_sp, csrc_sp] if first_layer else [den_sp]):
            @pl.when(s < NS - 1)
            def _(sp=sp):
                pltpu.sync_copy(d1.at[pl.ds(0, SEG_FULL)],
                                sp.at[pl.ds(st, SEG_FULL)])

            @pl.when(s == NS - 1)
            def _(sp=sp):
                pltpu.sync_copy(d1.at[pl.ds(0, SEG_LAST)],
                                sp.at[pl.ds(st, SEG_LAST)])
        plsc.subcore_barrier()

        pltpu.sync_copy(ai_hm.at[c], tbl_a)
        pltpu.sync_copy(aj_hm.at[c], tbl_b)
        pltpu.sync_copy(mh.at[c], mbuf)
        mvec = mbuf[...]

        def p1_chunk(ch, _):
            eb = (s * ROWS_PER_TILE + ch * CHUNK_ROWS) * BW
            pltpu.sync_copy(src1.at[pl.ds(eb, EDGES_PER_CHUNK)], sidx)
            pltpu.sync_copy(dst1.at[pl.ds(eb, EDGES_PER_CHUNK)], didx)

            @plsc.parallel_loop(0, EDGES_PER_CHUNK // LN, 1, unroll=4)
            def _(k):
                sl = pl.ds(k * LN, LN)
                av = plsc.load_gather(tbl_a, [sidx[sl]])
                bv = plsc.load_gather(tbl_b, [didx[sl]])
                logit = av + bv
                logit = jnp.maximum(logit, 0.2 * logit)
                ebuf[sl] = jnp.exp(logit - mvec)

            pltpu.sync_copy(ebuf, ex_o.at[c, pl.ds(eb, EDGES_PER_CHUNK)])
            pltpu.sync_copy(ebuf, den_sp.at[didx], add=True)
            if first_layer:
                pltpu.sync_copy(ones, cdst_sp.at[didx], add=True)
                pltpu.sync_copy(ones, csrc_sp.at[sidx], add=True)
            return _
        lax.fori_loop(0, NCHUNKS, p1_chunk, None)
        plsc.subcore_barrier()

        def p15(size):
            pltpu.sync_copy(den_sp.at[pl.ds(st, size)], d1.at[pl.ds(0, size)])
            if first_layer:
                pltpu.sync_copy(cdst_sp.at[pl.ds(st, size)],
                                d2.at[pl.ds(0, size)])
                pltpu.sync_copy(csrc_sp.at[pl.ds(st, size)],
                                d3.at[pl.ds(0, size)])

                @pl.when(c == 1)
                def _():
                    pltpu.sync_copy(d2.at[pl.ds(0, size)],
                                    cntdst_o.at[pl.ds(st, size)])
            else:
                pltpu.sync_copy(cntdst_in.at[pl.ds(st, size)],
                                d2.at[pl.ds(0, size)])

            def vec(k, _):
                sl = pl.ds(k * LN, LN)
                de = d1[sl] + _EPS
                d1[sl] = de
                d2[sl] = de * d2[sl]
                if first_layer:
                    cs = d3[sl]
                    d3[sl] = jnp.where(cs > 0.0, 1.0 / cs, 0.0)
                return _
            lax.fori_loop(0, size // LN, vec, None)

            pltpu.sync_copy(d1.at[pl.ds(0, size)], den_sp.at[pl.ds(st, size)])
            pltpu.sync_copy(d2.at[pl.ds(0, size)], cdst_sp.at[pl.ds(st, size)])
            if first_layer:
                @pl.when(c == 0)
                def _():
                    pltpu.sync_copy(d3.at[pl.ds(0, size)],
                                    drec_o.at[pl.ds(st, size)])

        @pl.when(s < NS - 1)
        def _():
            p15(SEG_FULL)

        @pl.when(s == NS - 1)
        def _():
            p15(SEG_LAST)
        plsc.subcore_barrier()

        pltpu.sync_copy(den_sp, tbl_a)
        pltpu.sync_copy(cdst_sp, tbl_b)

        def p2_chunk(ch, _):
            eb = (s * ROWS_PER_TILE + ch * CHUNK_ROWS) * BW
            pltpu.sync_copy(dst1.at[pl.ds(eb, EDGES_PER_CHUNK)], didx)
            pltpu.sync_copy(ex_o.at[c, pl.ds(eb, EDGES_PER_CHUNK)], ebuf)

            @plsc.parallel_loop(0, EDGES_PER_CHUNK // LN, 1, unroll=4)
            def _(k):
                sl = pl.ds(k * LN, LN)
                jv = didx[sl]
                exv = ebuf[sl]
                d0 = plsc.load_gather(tbl_a, [jv])
                dd1 = plsc.load_gather(tbl_b, [jv])
                ebuf[sl] = exv / d0
                a1buf[sl] = exv / dd1
            pltpu.sync_copy(ebuf, alpha_o.at[c, pl.ds(eb, EDGES_PER_CHUNK)])
            pltpu.sync_copy(a1buf, alpha1_o.at[c, pl.ds(eb, EDGES_PER_CHUNK)])
            return _
        lax.fori_loop(0, NCHUNKS, p2_chunk, None)

    cparams = pltpu.CompilerParams(use_tc_tiling_on_sc=False,
                                   needs_layout_passes=False)
    if first_layer:
        def call(src1, dst1, ai_hm, aj_hm, mh):
            def body0(src1, dst1, ai_hm, aj_hm, mh, *rest):
                return body(src1, dst1, ai_hm, aj_hm, mh, None, *rest)
            fn = pl.kernel(body0, out_type=out_type, mesh=mesh,
                           scratch_types=list(scratch.values()),
                           compiler_params=cparams)
            return fn(src1, dst1, ai_hm, aj_hm, mh)
        return call
    else:
        def call(src1, dst1, ai_hm, aj_hm, mh, cntdst):
            def body1(src1, dst1, ai_hm, aj_hm, mh, cntdst_in,
                      ex_o, alpha_o, alpha1_o, *rest):
                return body(src1, dst1, ai_hm, aj_hm, mh, cntdst_in,
                            ex_o, alpha_o, alpha1_o, None, None, *rest)
            fn = pl.kernel(body1, out_type=out_type, mesh=mesh,
                           scratch_types=list(scratch.values()),
                           compiler_params=cparams)
            return fn(src1, dst1, ai_hm, aj_hm, mh, cntdst)
        return call


def _make_sc_b(first_layer):
    mesh = plsc.VectorSubcoreMesh(core_axis_name="c", subcore_axis_name="s",
                                  num_cores=NC, num_subcores=NS)
    out_type = [
        jax.ShapeDtypeStruct((HEADS, N_NODES, OUT), _f32),
        jax.ShapeDtypeStruct((HEADS, N_HE, OUT), _f32),
    ]
    scratch = [
        pltpu.VMEM((2, EPB, OUT), _f32),
        pltpu.VMEM((2, EPB), jnp.int32),
        pltpu.VMEM((2, EPB), jnp.int32),
        pltpu.VMEM((2, EPB), _f32),
        pltpu.VMEM((OCH,), _f32),
        pltpu.VMEM((OUT,), _f32),
        pltpu.VMEM_SHARED((N_HE, OUT), _f32),
        pltpu.SemaphoreType.DMA,
        pltpu.SemaphoreType.DMA,
        pltpu.SemaphoreType.DMA,
    ]

    def body(src1, dst1, xh_hm, alpha, alpha1, drec, bias_hm,
             h_o, oute_o,
             rowbuf, sidx, didx, abuf, dbuf, biasv, acc,
             semG, semS, semL):
        c = lax.axis_index("c")
        s = lax.axis_index("s")

        def zrows(h):
            def zr(r, _):
                rowbuf[h, r, pl.ds(0, LN)] = jnp.zeros((LN,), _f32)
                rowbuf[h, r, pl.ds(LN, LN)] = jnp.zeros((LN,), _f32)
                return _
            lax.fori_loop(0, EPB, zr, None)

        zrows(0)
        zrows(1)
        st = s * SEG_FULL

        def zseg(total):
            o = 0
            while o < total:
                nsz = min(EPB, total - o)
                if nsz == EPB:
                    pltpu.sync_copy(rowbuf.at[0], acc.at[pl.ds(st + o, nsz)])
                else:
                    pltpu.sync_copy(rowbuf.at[0, pl.ds(0, nsz)],
                                    acc.at[pl.ds(st + o, nsz)])
                o += nsz

        @pl.when(s < NS - 1)
        def _():
            zseg(SEG_FULL)

        @pl.when(s == NS - 1)
        def _():
            zseg(SEG_LAST)
        plsc.subcore_barrier()

        def propagate(table_hbm, gidx, scatteridx, wsrc, zero_set1):
            tbl = table_hbm.at[c]

            def ebase(ch):
                return (s * ROWS_PER_TILE + ch * CHB) * BW

            def g_of(h):
                return sidx.at[h] if gidx == "src" else didx.at[h]

            def sc_of(h):
                return sidx.at[h] if scatteridx == "src" else didx.at[h]

            def issue_loads(ch, h):
                eb = ebase(ch)
                pltpu.async_copy(src1.at[pl.ds(eb, EPB)], sidx.at[h], semL)
                pltpu.async_copy(dst1.at[pl.ds(eb, EPB)], didx.at[h], semL)
                pltpu.async_copy(wsrc.at[c, pl.ds(eb, EPB)], abuf.at[h], semL)

            def wait_loads(h):
                pltpu.make_async_copy(src1.at[pl.ds(0, EPB)], sidx.at[h],
                                      semL).wait()
                pltpu.make_async_copy(dst1.at[pl.ds(0, EPB)], didx.at[h],
                                      semL).wait()
                pltpu.make_async_copy(wsrc.at[c, pl.ds(0, EPB)], abuf.at[h],
                                      semL).wait()

            def issue_gather(h):
                pltpu.async_copy(tbl.at[g_of(h)], rowbuf.at[h], semG)

            def wait_gather(h):
                pltpu.make_async_copy(tbl.at[g_of(h)], rowbuf.at[h],
                                      semG).wait()

            def issue_scatter(h):
                pltpu.async_copy(rowbuf.at[h], acc.at[sc_of(h)], semS,
                                 add=True)

            def wait_scatter(h):
                pltpu.make_async_copy(rowbuf.at[h], acc.at[sc_of(h)],
                                      semS).wait()

            def compute(h):
                @plsc.parallel_loop(0, EPB // LN, 1, unroll=8)
                def _(q):
                    for i in range(LN):
                        e = q * LN + i
                        w = plsc.load_gather(
                            abuf.at[h], [jnp.full((LN,), e, jnp.int32)])
                        rowbuf[h, e, pl.ds(0, LN)] = \
                            rowbuf[h, e, pl.ds(0, LN)] * w
                        rowbuf[h, e, pl.ds(LN, LN)] = \
                            rowbuf[h, e, pl.ds(LN, LN)] * w

            if zero_set1:
                zrows(1)
            pltpu.sync_copy(src1.at[pl.ds(ebase(0), EPB)], sidx.at[1])
            pltpu.sync_copy(dst1.at[pl.ds(ebase(0), EPB)], didx.at[1])
            issue_scatter(1)
            issue_loads(0, 0)
            wait_loads(0)
            issue_gather(0)

            def pair(i, _):
                for h in (0, 1):
                    ch = 2 * i + h
                    wait_scatter(1 - h)
                    issue_loads(ch + 1, 1 - h)
                    wait_gather(h)
                    wait_loads(1 - h)
                    issue_gather(1 - h)
                    compute(h)
                    issue_scatter(h)
                return _
            lax.fori_loop(0, (NCHB - 1) // 2, pair, None)

            wait_scatter(1)
            wait_gather(0)
            compute(0)
            issue_scatter(0)
            wait_scatter(0)

        propagate(xh_hm, "src", "dst", alpha1, zero_set1=False)
        plsc.subcore_barrier()

        zrows(1)
        r0t = s * OTILE

        def wout(nchunks):
            for k in range(nchunks):
                r0 = r0t + k * OCH
                pltpu.sync_copy(acc.at[pl.ds(r0, OCH)], rowbuf.at[0])
                pltpu.sync_copy(rowbuf.at[0], oute_o.at[c, pl.ds(r0, OCH)])
                pltpu.sync_copy(rowbuf.at[1], acc.at[pl.ds(r0, OCH)])

        @pl.when(s < NS - 1)
        def _():
            wout(OTILE // OCH)

        @pl.when(s == NS - 1)
        def _():
            wout(OLAST // OCH)
        plsc.subcore_barrier()

        propagate(oute_o, "dst", "src", alpha, zero_set1=True)
        plsc.subcore_barrier()

        pltpu.sync_copy(bias_hm.at[c], biasv)
        blo = biasv[pl.ds(0, LN)]
        bhi = biasv[pl.ds(LN, LN)]

        def fin(nchunks):
            for k in range(nchunks):
                r0 = r0t + k * OCH
                pltpu.sync_copy(acc.at[pl.ds(r0, OCH)], rowbuf.at[0])
                pltpu.sync_copy(drec.at[pl.ds(r0, OCH)], dbuf)

                @plsc.parallel_loop(0, OCH, 1, unroll=4)
                def _(r):
                    dv = plsc.load_gather(dbuf,
                                          [jnp.full((LN,), r, jnp.int32)])
                    lo = rowbuf[0, r, pl.ds(0, LN)] * dv + blo
                    hi = rowbuf[0, r, pl.ds(LN, LN)] * dv + bhi
                    if first_layer:
                        lo = jnp.maximum(lo, 0.0)
                        hi = jnp.maximum(hi, 0.0)
                    rowbuf[0, r, pl.ds(0, LN)] = lo
                    rowbuf[0, r, pl.ds(LN, LN)] = hi
                pltpu.sync_copy(rowbuf.at[0], h_o.at[c, pl.ds(r0, OCH)])

        @pl.when(s < NS - 1)
        def _():
            fin(OTILE // OCH)

        @pl.when(s == NS - 1)
        def _():
            fin(OLAST // OCH)

    return pl.kernel(body, out_type=out_type, mesh=mesh,
                     scratch_types=scratch,
                     compiler_params=pltpu.CompilerParams(
                         use_tc_tiling_on_sc=False,
                         needs_layout_passes=False))


def _att_mats(att):
    a = att[0]
    Ai = jnp.zeros((D_IN, HEADS), _f32)
    Aj = jnp.zeros((D_IN, HEADS), _f32)
    for h in range(HEADS):
        Ai = Ai.at[h * OUT:(h + 1) * OUT, h].set(a[h, :OUT])
        Aj = Aj.at[h * OUT:(h + 1) * OUT, h].set(a[h, OUT:])
    return Ai, Aj


@jax.jit
def kernel(x, hyperedges, hyperedge_attr, W0, att0, b0, W1, att1, b1):
    src1 = hyperedges[0]
    dst1 = hyperedges[1]

    sc_a0 = _make_sc_a(True)
    sc_a1 = _make_sc_a(False)
    sc_b0 = _make_sc_b(True)
    sc_b1 = _make_sc_b(False)

    def layer(F, W, att, b, sc_a, sc_b, extra):
        Ai, Aj = _att_mats(att)
        xh_hm, ai, aj, m4 = _tc_embed(F, hyperedge_attr, W, Ai, Aj)
        ai_hm = ai.T
        aj_hm = aj.T
        m = m4[:2] + m4[2:]
        m = jnp.maximum(m, 0.2 * m)
        mh = jnp.tile(m[:, None], (1, LN))
        if extra is None:
            _ex, alpha, alpha1, drec, cntdst = sc_a(
                src1, dst1, ai_hm, aj_hm, mh)
        else:
            drec, cntdst = extra
            _ex, alpha, alpha1 = sc_a(src1, dst1, ai_hm, aj_hm, mh, cntdst)
        bias_hm = b.reshape(HEADS, OUT)
        h_hm, _oute = sc_b(src1, dst1, xh_hm, alpha, alpha1, drec, bias_hm)
        return h_hm, (drec, cntdst)

    h_hm, dd = layer(x, W0, att0, b0, sc_a0, sc_b0, None)
    h64 = h_hm.transpose(1, 0, 2).reshape(N_NODES, HEADS * OUT)
    out_hm, _ = layer(h64, W1, att1, b1, sc_a1, sc_b1, dd)
    return out_hm.transpose(1, 0, 2).reshape(N_NODES, HEADS * OUT)
